# baseline (speedup 1.0000x reference)
import jax
import jax.numpy as jnp
from jax.experimental import pallas as pl
from jax.experimental.pallas import tpu as pltpu

_BLOCK_R = 4096


def _body(ptr_ref, x_ref, w_ref, b_ref, o_ref):
    i = pl.program_id(0)
    r = x_ref.shape[0]
    nseg = o_ref.shape[0]
    first = ptr_ref[0] // r
    last = jnp.maximum(ptr_ref[nseg] - 1, ptr_ref[0]) // r
    j = first + i

    @pl.when(i == 0)
    def _init():
        o_ref[...] = jnp.zeros_like(o_ref)

    @pl.when(j <= last)
    def _acc():
        h = jax.lax.dot_general(
            x_ref[...].astype(jnp.bfloat16), w_ref[...].astype(jnp.bfloat16),
            (((1,), (1,)), ((), ())),
            preferred_element_type=jnp.float32)
        h = h + b_ref[...]
        h = jnp.maximum(h, 0.2 * h)
        lo = jnp.stack(
            [ptr_ref[s] for s in range(nseg)]).reshape(nseg, 1)
        hi = jnp.stack(
            [ptr_ref[s + 1] for s in range(nseg)]).reshape(nseg, 1)
        pos = j * r + jax.lax.broadcasted_iota(jnp.int32, (nseg, r), 1)
        sel = ((pos >= lo) & (pos < hi)).astype(jnp.bfloat16)
        o_ref[...] += jnp.dot(sel, h.astype(jnp.bfloat16),
                              preferred_element_type=jnp.float32)


def kernel(h_dag, obs_ptr, W, b):
    n, d = h_dag.shape
    nseg = obs_ptr.shape[0] - 1
    r = _BLOCK_R

    def x_map(i, ptr):
        first = ptr[0] // r
        last = jnp.maximum(ptr[nseg] - 1, ptr[0]) // r
        return (jnp.minimum(first + i, last), 0)

    grid_spec = pltpu.PrefetchScalarGridSpec(
        num_scalar_prefetch=1,
        grid=(n // r,),
        in_specs=[
            pl.BlockSpec((r, d), x_map),
            pl.BlockSpec((d, d), lambda i, ptr: (0, 0)),
            pl.BlockSpec((d,), lambda i, ptr: (0,)),
        ],
        out_specs=pl.BlockSpec((nseg, d), lambda i, ptr: (0, 0)),
    )
    return pl.pallas_call(
        _body,
        grid_spec=grid_spec,
        out_shape=jax.ShapeDtypeStruct((nseg, d), jnp.float32),
    )(obs_ptr, h_dag, W, b)


# speedup vs baseline: 1.1091x; 1.1091x over previous
import jax
import jax.numpy as jnp
from jax.experimental import pallas as pl
from jax.experimental.pallas import tpu as pltpu

_BLOCK_R = 8192


def _body(ptr_ref, x_ref, w_ref, b_ref, o_ref):
    i = pl.program_id(0)
    r = x_ref.shape[0]
    nseg = o_ref.shape[0]
    first = ptr_ref[0] // r
    last = jnp.maximum(ptr_ref[nseg] - 1, ptr_ref[0]) // r
    j = first + i

    @pl.when(i == 0)
    def _init():
        o_ref[...] = jnp.zeros_like(o_ref)

    @pl.when(j <= last)
    def _acc():
        h = jax.lax.dot_general(
            x_ref[...].astype(jnp.bfloat16), w_ref[...].astype(jnp.bfloat16),
            (((1,), (1,)), ((), ())),
            preferred_element_type=jnp.float32)
        h = h + b_ref[...]
        h = jnp.maximum(h, 0.2 * h)
        lo = jnp.stack(
            [ptr_ref[s] for s in range(nseg)]).reshape(nseg, 1)
        hi = jnp.stack(
            [ptr_ref[s + 1] for s in range(nseg)]).reshape(nseg, 1)
        pos = j * r + jax.lax.broadcasted_iota(jnp.int32, (nseg, r), 1)
        sel = ((pos >= lo) & (pos < hi)).astype(jnp.bfloat16)
        o_ref[...] += jnp.dot(sel, h.astype(jnp.bfloat16),
                              preferred_element_type=jnp.float32)


def kernel(h_dag, obs_ptr, W, b):
    n, d = h_dag.shape
    nseg = obs_ptr.shape[0] - 1
    r = _BLOCK_R

    def x_map(i, ptr):
        first = ptr[0] // r
        last = jnp.maximum(ptr[nseg] - 1, ptr[0]) // r
        return (jnp.minimum(first + i, last), 0)

    grid_spec = pltpu.PrefetchScalarGridSpec(
        num_scalar_prefetch=1,
        grid=(n // r,),
        in_specs=[
            pl.BlockSpec((r, d), x_map),
            pl.BlockSpec((d, d), lambda i, ptr: (0, 0)),
            pl.BlockSpec((d,), lambda i, ptr: (0,)),
        ],
        out_specs=pl.BlockSpec((nseg, d), lambda i, ptr: (0, 0)),
    )
    return pl.pallas_call(
        _body,
        grid_spec=grid_spec,
        out_shape=jax.ShapeDtypeStruct((nseg, d), jnp.float32),
    )(obs_ptr, h_dag, W, b)


# speedup vs baseline: 1.1307x; 1.0195x over previous
import jax
import jax.numpy as jnp
from jax.experimental import pallas as pl
from jax.experimental.pallas import tpu as pltpu

_BLOCK_R = 8192


def _body(ptr_ref, x_ref, w_ref, b_ref, o_ref):
    i = pl.program_id(0)
    r = x_ref.shape[0]
    nseg = o_ref.shape[0]
    first = ptr_ref[0] // r
    last = jnp.maximum(ptr_ref[nseg] - 1, ptr_ref[0]) // r
    j = first + i

    @pl.when(i == 0)
    def _init():
        o_ref[...] = jnp.zeros_like(o_ref)

    @pl.when(j <= last)
    def _acc():
        h = jax.lax.dot_general(
            x_ref[...].astype(jnp.bfloat16), w_ref[...].astype(jnp.bfloat16),
            (((1,), (1,)), ((), ())),
            preferred_element_type=jnp.float32).astype(jnp.bfloat16)
        h = h + b_ref[...].astype(jnp.bfloat16)
        h = jnp.maximum(h, jnp.bfloat16(0.2) * h)
        lo = jnp.stack(
            [ptr_ref[s] for s in range(nseg)]).reshape(nseg, 1)
        hi = jnp.stack(
            [ptr_ref[s + 1] for s in range(nseg)]).reshape(nseg, 1)
        pos = j * r + jax.lax.broadcasted_iota(jnp.int32, (nseg, r), 1)
        sel = ((pos >= lo) & (pos < hi)).astype(jnp.bfloat16)
        o_ref[...] += jnp.dot(sel, h.astype(jnp.bfloat16),
                              preferred_element_type=jnp.float32)


def kernel(h_dag, obs_ptr, W, b):
    n, d = h_dag.shape
    nseg = obs_ptr.shape[0] - 1
    r = _BLOCK_R

    def x_map(i, ptr):
        first = ptr[0] // r
        last = jnp.maximum(ptr[nseg] - 1, ptr[0]) // r
        return (jnp.minimum(first + i, last), 0)

    grid_spec = pltpu.PrefetchScalarGridSpec(
        num_scalar_prefetch=1,
        grid=(n // r,),
        in_specs=[
            pl.BlockSpec((r, d), x_map),
            pl.BlockSpec((d, d), lambda i, ptr: (0, 0)),
            pl.BlockSpec((d,), lambda i, ptr: (0,)),
        ],
        out_specs=pl.BlockSpec((nseg, d), lambda i, ptr: (0, 0)),
    )
    return pl.pallas_call(
        _body,
        grid_spec=grid_spec,
        out_shape=jax.ShapeDtypeStruct((nseg, d), jnp.float32),
    )(obs_ptr, h_dag, W, b)


# speedup vs baseline: 1.1335x; 1.0024x over previous
import jax
import jax.numpy as jnp
from jax.experimental import pallas as pl
from jax.experimental.pallas import tpu as pltpu

_BLOCK_R = 8192


def _body(ptr_ref, x_ref, w_ref, b_ref, o_ref):
    i = pl.program_id(0)
    r = x_ref.shape[0]
    nseg = o_ref.shape[0]
    first = ptr_ref[0] // r
    last = jnp.maximum(ptr_ref[nseg] - 1, ptr_ref[0]) // r
    j = first + i

    @pl.when(i == 0)
    def _init():
        o_ref[...] = jnp.zeros_like(o_ref)

    @pl.when(j <= last)
    def _acc():
        h = jax.lax.dot_general(
            x_ref[...].astype(jnp.bfloat16), w_ref[...].astype(jnp.bfloat16),
            (((1,), (1,)), ((), ())),
            preferred_element_type=jnp.float32).astype(jnp.bfloat16)
        h = h + b_ref[...].astype(jnp.bfloat16)
        h = jnp.maximum(h, jnp.bfloat16(0.2) * h)
        lo = jnp.stack(
            [ptr_ref[s] for s in range(nseg)]).reshape(nseg, 1)
        hi = jnp.stack(
            [ptr_ref[s + 1] for s in range(nseg)]).reshape(nseg, 1)
        pos = j * r + jax.lax.broadcasted_iota(jnp.int32, (nseg, r), 1)
        sel = ((pos >= lo) & (pos < hi)).astype(jnp.bfloat16)
        o_ref[...] += jnp.dot(sel, h, preferred_element_type=jnp.float32)


def kernel(h_dag, obs_ptr, W, b):
    n, d = h_dag.shape
    nseg = obs_ptr.shape[0] - 1
    r = _BLOCK_R

    def x_map(i, ptr):
        first = ptr[0] // r
        last = jnp.maximum(ptr[nseg] - 1, ptr[0]) // r
        return (jnp.minimum(first + i, last), 0)

    grid_spec = pltpu.PrefetchScalarGridSpec(
        num_scalar_prefetch=1,
        grid=(n // r,),
        in_specs=[
            pl.BlockSpec((r, d), x_map),
            pl.BlockSpec((d, d), lambda i, ptr: (0, 0)),
            pl.BlockSpec((d,), lambda i, ptr: (0,)),
        ],
        out_specs=pl.BlockSpec((nseg, d), lambda i, ptr: (0, 0)),
    )
    return pl.pallas_call(
        _body,
        grid_spec=grid_spec,
        out_shape=jax.ShapeDtypeStruct((nseg, d), jnp.float32),
    )(obs_ptr, h_dag, W, b)
